# baseline (speedup 1.0000x reference)
import jax
import jax.numpy as jnp
from jax import lax
from jax.experimental import pallas as pl
from jax.experimental.pallas import tpu as pltpu

_N = 320000
_DI = 125
_DP = 3
_DO = 128
_NSEG = 16
_B = 12800
_NBLK = _N // _B
_IMAX = jnp.iinfo(jnp.int32).max


def _seg_kernel(batch_smem, x_ref, pos_ref, batch_ref, w1_ref, w2_ref, b_ref,
                out_ref, qidx_ref):
    i = pl.program_id(0)

    @pl.when(i == 0)
    def _init():
        out_ref[...] = jnp.full((_NSEG, _DO), -jnp.inf, jnp.float32)
        qidx_ref[...] = jnp.full((_NSEG, _DO), _IMAX, jnp.int32)

    h = (jnp.dot(x_ref[...], w1_ref[...], preferred_element_type=jnp.float32)
         + lax.dot_general(pos_ref[...], w2_ref[...],
                           dimension_numbers=(((0,), (0,)), ((), ())),
                           preferred_element_type=jnp.float32))

    first = batch_smem[0, 0, 0]
    last = batch_smem[0, 0, _B - 1]
    base = i * _B

    seg_rows = lax.broadcasted_iota(jnp.int32, (_NSEG, _DO), 0)

    @pl.when(first == last)
    def _fast():
        mx8 = jnp.max(h.reshape(_B // 8, 8, _DO), axis=0)
        mx = jnp.max(mx8, axis=0, keepdims=True)
        sel = seg_rows == first
        out_ref[...] = jnp.where(sel, jnp.maximum(out_ref[...], mx), out_ref[...])
        qidx_ref[...] = jnp.where(sel, jnp.minimum(qidx_ref[...], base), qidx_ref[...])

    @pl.when(first != last)
    def _slow():
        bb = batch_ref[...].reshape(1, _B)
        lane_g = lax.broadcasted_iota(jnp.int32, (1, _B), 1) + base
        row_g = lax.broadcasted_iota(jnp.int32, (_B, 1), 0) + base

        def body(s, carry):
            m = bb == s
            start = jnp.min(jnp.where(m, lane_g, _IMAX), axis=1, keepdims=True)
            end = jnp.max(jnp.where(m, lane_g, -1), axis=1, keepdims=True)
            rmask = (row_g >= start) & (row_g <= end)
            mx = jnp.max(jnp.where(rmask, h, -jnp.inf), axis=0, keepdims=True)
            sel = seg_rows == s
            out_ref[...] = jnp.where(sel, jnp.maximum(out_ref[...], mx), out_ref[...])
            qidx_ref[...] = jnp.where(sel, jnp.minimum(qidx_ref[...], start), qidx_ref[...])
            return carry

        lax.fori_loop(first, last + 1, body, 0)

    @pl.when(i == _NBLK - 1)
    def _bias():
        out_ref[...] = out_ref[...] + b_ref[...]


def kernel(x, pos, batch, W, b):
    batch3 = batch.reshape(_NBLK, 1, _B)
    pos_t = pos.T
    w1 = W[:_DI]
    w2 = W[_DI:]
    b2 = b.reshape(1, _DO)

    pooled, qidx2 = pl.pallas_call(
        _seg_kernel,
        grid=(_NBLK,),
        in_specs=[
            pl.BlockSpec((1, 1, _B), lambda i: (i, 0, 0), memory_space=pltpu.SMEM),
            pl.BlockSpec((_B, _DI), lambda i: (i, 0)),
            pl.BlockSpec((_DP, _B), lambda i: (0, i)),
            pl.BlockSpec((1, 1, _B), lambda i: (i, 0, 0)),
            pl.BlockSpec((_DI, _DO), lambda i: (0, 0)),
            pl.BlockSpec((_DP, _DO), lambda i: (0, 0)),
            pl.BlockSpec((1, _DO), lambda i: (0, 0)),
        ],
        out_specs=[
            pl.BlockSpec((_NSEG, _DO), lambda i: (0, 0)),
            pl.BlockSpec((_NSEG, _DO), lambda i: (0, 0)),
        ],
        out_shape=[
            jax.ShapeDtypeStruct((_NSEG, _DO), jnp.float32),
            jax.ShapeDtypeStruct((_NSEG, _DO), jnp.int32),
        ],
        compiler_params=pltpu.CompilerParams(
            dimension_semantics=("arbitrary",),
        ),
    )(batch3, x, pos_t, batch3, w1, w2, b2)

    qidx = qidx2[:, 0]
    return pooled, pos[qidx], batch[qidx]


# speedup vs baseline: 1.3239x; 1.3239x over previous
import jax
import jax.numpy as jnp
from jax import lax
from jax.experimental import pallas as pl
from jax.experimental.pallas import tpu as pltpu

_N = 320000
_DI = 125
_DP = 3
_DO = 128
_NSEG = 16
_B = 6400
_G = _B // 8
_NBLK = _N // _B
_IMAX = jnp.iinfo(jnp.int32).max


def _seg_kernel(batch_smem, x_ref, pos_ref, w1_ref, w2_ref, b_ref,
                out_ref, qidx_ref, h_ref):
    i = pl.program_id(0)

    @pl.when(i == 0)
    def _init():
        out_ref[...] = jnp.full((_NSEG, _DO), -jnp.inf, jnp.float32)
        qidx_ref[...] = jnp.full((_NSEG, _DO), _IMAX, jnp.int32)

    h_ref[...] = (
        jnp.dot(x_ref[...], w1_ref[...], preferred_element_type=jnp.float32)
        + lax.dot_general(pos_ref[...], w2_ref[...],
                          dimension_numbers=(((0,), (0,)), ((), ())),
                          preferred_element_type=jnp.float32))

    g8 = jnp.max(h_ref[...].reshape(_G, 8, _DO), axis=1)

    first = batch_smem[0, 0, 0]
    last = batch_smem[0, 0, _B - 1]
    base = i * _B

    seg_rows = lax.broadcasted_iota(jnp.int32, (_NSEG, _DO), 0)

    @pl.when(first == last)
    def _fast():
        mx = jnp.max(g8, axis=0, keepdims=True)
        sel = seg_rows == first
        out_ref[...] = jnp.where(sel, jnp.maximum(out_ref[...], mx), out_ref[...])
        qidx_ref[...] = jnp.where(sel, jnp.minimum(qidx_ref[...], base), qidx_ref[...])

    @pl.when(first != last)
    def _slow():
        grp_start = lax.broadcasted_iota(jnp.int32, (_G, 1), 0) * 8
        row8 = lax.broadcasted_iota(jnp.int32, (8, 1), 0)

        def lower_bound(v):
            def cond(c):
                return c[0] < c[1]

            def step(c):
                lo, hi = c
                mid = (lo + hi) // 2
                go_right = batch_smem[0, 0, mid] < v
                return (jnp.where(go_right, mid + 1, lo),
                        jnp.where(go_right, hi, mid))

            return lax.while_loop(cond, step, (0, _B))[0]

        def body(s, carry):
            start = lower_bound(s)
            end = lower_bound(s + 1) - 1
            gin = (grp_start >= start) & (grp_start + 7 <= end)
            mx = jnp.max(jnp.where(gin, g8, -jnp.inf), axis=0, keepdims=True)
            for q in (jnp.clip(start // 8, 0, _G - 1), jnp.clip(end // 8, 0, _G - 1)):
                rows = h_ref[pl.ds(q * 8, 8), :]
                rid = q * 8 + row8
                pm = (rid >= start) & (rid <= end)
                mx = jnp.maximum(mx, jnp.max(jnp.where(pm, rows, -jnp.inf),
                                             axis=0, keepdims=True))
            sel = seg_rows == s
            cand = jnp.where(start <= end, base + start, _IMAX)
            out_ref[...] = jnp.where(sel, jnp.maximum(out_ref[...], mx), out_ref[...])
            qidx_ref[...] = jnp.where(sel, jnp.minimum(qidx_ref[...], cand), qidx_ref[...])
            return carry

        lax.fori_loop(first, last + 1, body, 0)

    @pl.when(i == _NBLK - 1)
    def _bias():
        out_ref[...] = out_ref[...] + b_ref[...]


def kernel(x, pos, batch, W, b):
    batch3 = batch.reshape(_NBLK, 1, _B)
    pos_t = pos.T
    w1 = W[:_DI]
    w2 = W[_DI:]
    b2 = b.reshape(1, _DO)

    pooled, qidx2 = pl.pallas_call(
        _seg_kernel,
        grid=(_NBLK,),
        in_specs=[
            pl.BlockSpec((1, 1, _B), lambda i: (i, 0, 0), memory_space=pltpu.SMEM),
            pl.BlockSpec((_B, _DI), lambda i: (i, 0)),
            pl.BlockSpec((_DP, _B), lambda i: (0, i)),
            pl.BlockSpec((_DI, _DO), lambda i: (0, 0)),
            pl.BlockSpec((_DP, _DO), lambda i: (0, 0)),
            pl.BlockSpec((1, _DO), lambda i: (0, 0)),
        ],
        out_specs=[
            pl.BlockSpec((_NSEG, _DO), lambda i: (0, 0)),
            pl.BlockSpec((_NSEG, _DO), lambda i: (0, 0)),
        ],
        out_shape=[
            jax.ShapeDtypeStruct((_NSEG, _DO), jnp.float32),
            jax.ShapeDtypeStruct((_NSEG, _DO), jnp.int32),
        ],
        scratch_shapes=[pltpu.VMEM((_B, _DO), jnp.float32)],
        compiler_params=pltpu.CompilerParams(
            dimension_semantics=("arbitrary",),
        ),
    )(batch3, x, pos_t, w1, w2, b2)

    qidx = qidx2[:, 0]
    return pooled, pos[qidx], batch[qidx]


# speedup vs baseline: 1.4276x; 1.0784x over previous
import jax
import jax.numpy as jnp
from jax import lax
from jax.experimental import pallas as pl
from jax.experimental.pallas import tpu as pltpu

_N = 320000
_DI = 125
_DP = 3
_DO = 128
_NSEG = 16
_B = 12800
_G = _B // 8
_NBLK = _N // _B
_IMAX = jnp.iinfo(jnp.int32).max


def _seg_kernel(batch_smem, x_ref, pos_ref, w1_ref, w2_ref, b_ref,
                out_ref, qidx_ref, h_ref):
    i = pl.program_id(0)

    @pl.when(i == 0)
    def _init():
        out_ref[...] = jnp.full((_NSEG, _DO), -jnp.inf, jnp.float32)
        qidx_ref[...] = jnp.full((_NSEG, _DO), _IMAX, jnp.int32)

    h_ref[...] = (
        jnp.dot(x_ref[...], w1_ref[...], preferred_element_type=jnp.float32)
        + lax.dot_general(pos_ref[...], w2_ref[...],
                          dimension_numbers=(((0,), (0,)), ((), ())),
                          preferred_element_type=jnp.float32))

    g8 = jnp.max(h_ref[...].reshape(_G, 8, _DO), axis=1)

    first = batch_smem[0, 0, 0]
    last = batch_smem[0, 0, _B - 1]
    base = i * _B

    seg_rows = lax.broadcasted_iota(jnp.int32, (_NSEG, _DO), 0)

    @pl.when(first == last)
    def _fast():
        mx = jnp.max(g8, axis=0, keepdims=True)
        sel = seg_rows == first
        out_ref[...] = jnp.where(sel, jnp.maximum(out_ref[...], mx), out_ref[...])
        qidx_ref[...] = jnp.where(sel, jnp.minimum(qidx_ref[...], base), qidx_ref[...])

    @pl.when(first != last)
    def _slow():
        grp_start = lax.broadcasted_iota(jnp.int32, (_G, 1), 0) * 8
        row8 = lax.broadcasted_iota(jnp.int32, (8, 1), 0)

        def lower_bound(v):
            def cond(c):
                return c[0] < c[1]

            def step(c):
                lo, hi = c
                mid = (lo + hi) // 2
                go_right = batch_smem[0, 0, mid] < v
                return (jnp.where(go_right, mid + 1, lo),
                        jnp.where(go_right, hi, mid))

            return lax.while_loop(cond, step, (0, _B))[0]

        def body(s, carry):
            start = lower_bound(s)
            end = lower_bound(s + 1) - 1
            gin = (grp_start >= start) & (grp_start + 7 <= end)
            mx = jnp.max(jnp.where(gin, g8, -jnp.inf), axis=0, keepdims=True)
            for q in (jnp.clip(start // 8, 0, _G - 1), jnp.clip(end // 8, 0, _G - 1)):
                rows = h_ref[pl.ds(q * 8, 8), :]
                rid = q * 8 + row8
                pm = (rid >= start) & (rid <= end)
                mx = jnp.maximum(mx, jnp.max(jnp.where(pm, rows, -jnp.inf),
                                             axis=0, keepdims=True))
            sel = seg_rows == s
            cand = jnp.where(start <= end, base + start, _IMAX)
            out_ref[...] = jnp.where(sel, jnp.maximum(out_ref[...], mx), out_ref[...])
            qidx_ref[...] = jnp.where(sel, jnp.minimum(qidx_ref[...], cand), qidx_ref[...])
            return carry

        lax.fori_loop(first, last + 1, body, 0)

    @pl.when(i == _NBLK - 1)
    def _bias():
        out_ref[...] = out_ref[...] + b_ref[...]


def kernel(x, pos, batch, W, b):
    batch3 = batch.reshape(_NBLK, 1, _B)
    pos_t = pos.T
    w1 = W[:_DI]
    w2 = W[_DI:]
    b2 = b.reshape(1, _DO)

    pooled, qidx2 = pl.pallas_call(
        _seg_kernel,
        grid=(_NBLK,),
        in_specs=[
            pl.BlockSpec((1, 1, _B), lambda i: (i, 0, 0), memory_space=pltpu.SMEM),
            pl.BlockSpec((_B, _DI), lambda i: (i, 0)),
            pl.BlockSpec((_DP, _B), lambda i: (0, i)),
            pl.BlockSpec((_DI, _DO), lambda i: (0, 0)),
            pl.BlockSpec((_DP, _DO), lambda i: (0, 0)),
            pl.BlockSpec((1, _DO), lambda i: (0, 0)),
        ],
        out_specs=[
            pl.BlockSpec((_NSEG, _DO), lambda i: (0, 0)),
            pl.BlockSpec((_NSEG, _DO), lambda i: (0, 0)),
        ],
        out_shape=[
            jax.ShapeDtypeStruct((_NSEG, _DO), jnp.float32),
            jax.ShapeDtypeStruct((_NSEG, _DO), jnp.int32),
        ],
        scratch_shapes=[pltpu.VMEM((_B, _DO), jnp.float32)],
        compiler_params=pltpu.CompilerParams(
            dimension_semantics=("arbitrary",),
        ),
    )(batch3, x, pos_t, w1, w2, b2)

    qidx = qidx2[:, 0]
    return pooled, pos[qidx], batch[qidx]


# speedup vs baseline: 1.4605x; 1.0231x over previous
import jax
import jax.numpy as jnp
from jax import lax
from jax.experimental import pallas as pl
from jax.experimental.pallas import tpu as pltpu

_N = 320000
_DI = 125
_DP = 3
_DO = 128
_NSEG = 16
_B = 16000
_G = _B // 8
_NBLK = _N // _B
_IMAX = jnp.iinfo(jnp.int32).max


def _seg_kernel(batch_smem, x_ref, pos_ref, w1_ref, w2_ref, b_ref,
                out_ref, qidx_ref, h_ref):
    i = pl.program_id(0)

    @pl.when(i == 0)
    def _init():
        out_ref[...] = jnp.full((_NSEG, _DO), -jnp.inf, jnp.float32)
        qidx_ref[...] = jnp.full((_NSEG, _DO), _IMAX, jnp.int32)

    h_ref[...] = (
        jnp.dot(x_ref[...], w1_ref[...], preferred_element_type=jnp.float32)
        + lax.dot_general(pos_ref[...], w2_ref[...],
                          dimension_numbers=(((0,), (0,)), ((), ())),
                          preferred_element_type=jnp.float32))

    g8 = jnp.max(h_ref[...].reshape(_G, 8, _DO), axis=1)

    first = batch_smem[0, 0, 0]
    last = batch_smem[0, 0, _B - 1]
    base = i * _B

    seg_rows = lax.broadcasted_iota(jnp.int32, (_NSEG, _DO), 0)

    @pl.when(first == last)
    def _fast():
        mx = jnp.max(g8, axis=0, keepdims=True)
        sel = seg_rows == first
        out_ref[...] = jnp.where(sel, jnp.maximum(out_ref[...], mx), out_ref[...])
        qidx_ref[...] = jnp.where(sel, jnp.minimum(qidx_ref[...], base), qidx_ref[...])

    @pl.when(first != last)
    def _slow():
        grp_start = lax.broadcasted_iota(jnp.int32, (_G, 1), 0) * 8
        row8 = lax.broadcasted_iota(jnp.int32, (8, 1), 0)

        def lower_bound(v):
            def cond(c):
                return c[0] < c[1]

            def step(c):
                lo, hi = c
                mid = (lo + hi) // 2
                go_right = batch_smem[0, 0, mid] < v
                return (jnp.where(go_right, mid + 1, lo),
                        jnp.where(go_right, hi, mid))

            return lax.while_loop(cond, step, (0, _B))[0]

        def body(s, carry):
            start = lower_bound(s)
            end = lower_bound(s + 1) - 1
            gin = (grp_start >= start) & (grp_start + 7 <= end)
            mx = jnp.max(jnp.where(gin, g8, -jnp.inf), axis=0, keepdims=True)
            for q in (jnp.clip(start // 8, 0, _G - 1), jnp.clip(end // 8, 0, _G - 1)):
                rows = h_ref[pl.ds(q * 8, 8), :]
                rid = q * 8 + row8
                pm = (rid >= start) & (rid <= end)
                mx = jnp.maximum(mx, jnp.max(jnp.where(pm, rows, -jnp.inf),
                                             axis=0, keepdims=True))
            sel = seg_rows == s
            cand = jnp.where(start <= end, base + start, _IMAX)
            out_ref[...] = jnp.where(sel, jnp.maximum(out_ref[...], mx), out_ref[...])
            qidx_ref[...] = jnp.where(sel, jnp.minimum(qidx_ref[...], cand), qidx_ref[...])
            return carry

        lax.fori_loop(first, last + 1, body, 0)

    @pl.when(i == _NBLK - 1)
    def _bias():
        out_ref[...] = out_ref[...] + b_ref[...]


def kernel(x, pos, batch, W, b):
    batch3 = batch.reshape(_NBLK, 1, _B)
    pos_t = pos.T
    w1 = W[:_DI]
    w2 = W[_DI:]
    b2 = b.reshape(1, _DO)

    pooled, qidx2 = pl.pallas_call(
        _seg_kernel,
        grid=(_NBLK,),
        in_specs=[
            pl.BlockSpec((1, 1, _B), lambda i: (i, 0, 0), memory_space=pltpu.SMEM),
            pl.BlockSpec((_B, _DI), lambda i: (i, 0)),
            pl.BlockSpec((_DP, _B), lambda i: (0, i)),
            pl.BlockSpec((_DI, _DO), lambda i: (0, 0)),
            pl.BlockSpec((_DP, _DO), lambda i: (0, 0)),
            pl.BlockSpec((1, _DO), lambda i: (0, 0)),
        ],
        out_specs=[
            pl.BlockSpec((_NSEG, _DO), lambda i: (0, 0)),
            pl.BlockSpec((_NSEG, _DO), lambda i: (0, 0)),
        ],
        out_shape=[
            jax.ShapeDtypeStruct((_NSEG, _DO), jnp.float32),
            jax.ShapeDtypeStruct((_NSEG, _DO), jnp.int32),
        ],
        scratch_shapes=[pltpu.VMEM((_B, _DO), jnp.float32)],
        compiler_params=pltpu.CompilerParams(
            dimension_semantics=("arbitrary",),
        ),
    )(batch3, x, pos_t, w1, w2, b2)

    qidx = qidx2[:, 0]
    return pooled, pos[qidx], batch[qidx]
